# Initial kernel scaffold; baseline (speedup 1.0000x reference)
import jax
import jax.numpy as jnp
from jax.experimental import pallas as pl


def kernel(x, edge_index, edge_attr, W_l, b_l, W_r, b_r, att, W_e, b_e, bias, gate_W, gate_b):
    raise NotImplementedError("write your pallas kernel here")


# speedup vs baseline: 8.8971x; 8.8971x over previous
import functools

import jax
import jax.numpy as jnp
from jax import lax
from jax.experimental import pallas as pl
from jax.experimental.pallas import tpu as pltpu
from jax.experimental.pallas import tpu_sc as plsc

N = 10000
E = 320000
IN = 128
OUT = 128
H = 4
ED = 16
NEG = 0.2
HO = H * OUT

NP = 10240
EP = 320256
EPA = 320512
NT = 32
NS = 16
EPT = EP // NT
C = 24
NCHUNK = EPT // C
ZR = NP // NS


def _proj2_body(x_ref, wl_ref, bl_ref, wr_ref, br_ref, xl_ref, xr_ref):
    xb = x_ref[...]
    xl_ref[...] = jnp.dot(xb, wl_ref[...], preferred_element_type=jnp.float32) + bl_ref[...]
    xr_ref[...] = jnp.dot(xb, wr_ref[...], preferred_element_type=jnp.float32) + br_ref[...]


def _proj2(xp, W_l, b_l, W_r, b_r):
    BM = 256
    grid = (NP // BM,)
    return pl.pallas_call(
        _proj2_body,
        grid=grid,
        in_specs=[
            pl.BlockSpec((BM, IN), lambda i: (i, 0)),
            pl.BlockSpec((IN, HO), lambda i: (0, 0)),
            pl.BlockSpec((1, HO), lambda i: (0, 0)),
            pl.BlockSpec((IN, HO), lambda i: (0, 0)),
            pl.BlockSpec((1, HO), lambda i: (0, 0)),
        ],
        out_specs=[
            pl.BlockSpec((BM, HO), lambda i: (i, 0)),
            pl.BlockSpec((BM, HO), lambda i: (i, 0)),
        ],
        out_shape=[
            jax.ShapeDtypeStruct((NP, HO), jnp.float32),
            jax.ShapeDtypeStruct((NP, HO), jnp.float32),
        ],
    )(xp, W_l, b_l.reshape(1, HO), W_r, b_r.reshape(1, HO))


def _proje_body(a_ref, w_ref, b_ref, o_ref):
    o_ref[...] = jnp.dot(a_ref[...], w_ref[...], preferred_element_type=jnp.float32) + b_ref[...]


def _proje(eap, W_e, b_e):
    BE = 1024
    grid = (EPA // BE,)
    return pl.pallas_call(
        _proje_body,
        grid=grid,
        in_specs=[
            pl.BlockSpec((BE, ED), lambda i: (i, 0)),
            pl.BlockSpec((ED, HO), lambda i: (0, 0)),
            pl.BlockSpec((1, HO), lambda i: (0, 0)),
        ],
        out_specs=pl.BlockSpec((BE, HO), lambda i: (i, 0)),
        out_shape=jax.ShapeDtypeStruct((EPA, HO), jnp.float32),
    )(eap, W_e, b_e.reshape(1, HO))


def _dencomb_body(dp_ref, o_ref):
    o_ref[...] = dp_ref[0] + dp_ref[1]


def _dencomb(den_part):
    return pl.pallas_call(
        _dencomb_body,
        out_shape=jax.ShapeDtypeStruct((NP, OUT), jnp.float32),
    )(den_part)


def _final_body(p_ref, b_ref, o_ref):
    o_ref[...] = p_ref[0] + p_ref[1] + b_ref[...]


def _final(out_part, bias):
    return pl.pallas_call(
        _final_body,
        out_shape=jax.ShapeDtypeStruct((NP, OUT), jnp.float32),
    )(out_part, bias.reshape(1, OUT))


def _hsum(v):
    iot = lax.iota(jnp.int32, 16)
    for s in (8, 4, 2, 1):
        v = v + v.at[iot ^ s].get(mode="promise_in_bounds")
    return v


def _bcast_lane(v, lane):
    idx = jnp.full((16,), lane, jnp.int32)
    return v.at[idx].get(mode="promise_in_bounds")


def _pass1_body(src_hbm, dst_hbm, xl_hbm, xr_hbm, ea_hbm, att_hbm, z_hbm,
                ex_hbm, den_hbm,
                src_v, dst_v, xl_v, xr_v, ea_v, ex_v, exd_v, att_v, den_sh,
                sem0, sem1):
    cid = lax.axis_index("c")
    sid = lax.axis_index("s")
    wid = cid * NS + sid

    pltpu.sync_copy(z_hbm.at[pl.ds(sid * ZR, ZR)], den_sh.at[pl.ds(sid * ZR, ZR)])
    pltpu.sync_copy(z_hbm.at[pl.ds(0, C)], exd_v)
    pltpu.sync_copy(att_hbm, att_v)
    plsc.subcore_barrier()

    iot = lax.iota(jnp.int32, 16)
    masks = [jnp.where(iot == h, 1.0, 0.0).astype(jnp.float32) for h in range(H)]
    base0 = wid * EPT

    @pl.loop(0, NCHUNK)
    def _chunk(k):
        base = base0 + k * C
        pltpu.sync_copy(src_hbm.at[pl.ds(base, C)], src_v)
        pltpu.sync_copy(dst_hbm.at[pl.ds(base, C)], dst_v)
        cp1 = pltpu.async_copy(xl_hbm.at[src_v], xl_v, sem0)
        cp2 = pltpu.async_copy(xr_hbm.at[dst_v], xr_v, sem1)
        pltpu.sync_copy(ea_hbm.at[pl.ds(base, C)], ea_v)
        cp1.wait()
        cp2.wait()

        @pl.loop(0, C)
        def _edge(i):
            lvec = jnp.zeros((16,), jnp.float32)
            for h in range(H):
                acc = jnp.zeros((16,), jnp.float32)
                for j in range(OUT // 16):
                    off = h * OUT + j * 16
                    v = (xl_v[i, pl.ds(off, 16)] + xr_v[i, pl.ds(off, 16)]
                         + ea_v[i, pl.ds(off, 16)])
                    v = jnp.maximum(v, v * NEG)
                    acc = acc + v * att_v[pl.ds(off, 16)]
                lvec = lvec + _hsum(acc) * masks[h]
            exv = jnp.exp(lvec)
            ex_v[i, :] = exv
            exd_v[i, pl.ds(0, 16)] = exv

        pltpu.sync_copy(ex_v, ex_hbm.at[pl.ds(base, C)])
        pltpu.sync_copy(exd_v, den_sh.at[dst_v], add=True)

    plsc.subcore_barrier()
    pltpu.sync_copy(den_sh.at[pl.ds(sid * ZR, ZR)],
                    den_hbm.at[cid, pl.ds(sid * ZR, ZR)])


def _pass1(src, dst, xl, xr, ea, att_flat, z128):
    mesh = plsc.VectorSubcoreMesh(core_axis_name="c", subcore_axis_name="s")
    fn = pl.kernel(
        _pass1_body,
        out_type=[
            jax.ShapeDtypeStruct((EP, 16), jnp.float32),
            jax.ShapeDtypeStruct((2, NP, OUT), jnp.float32),
        ],
        mesh=mesh,
        scratch_types=[
            pltpu.VMEM((C,), jnp.int32),
            pltpu.VMEM((C,), jnp.int32),
            pltpu.VMEM((C, HO), jnp.float32),
            pltpu.VMEM((C, HO), jnp.float32),
            pltpu.VMEM((C, HO), jnp.float32),
            pltpu.VMEM((C, 16), jnp.float32),
            pltpu.VMEM((C, OUT), jnp.float32),
            pltpu.VMEM((HO,), jnp.float32),
            pltpu.VMEM_SHARED((NP, OUT), jnp.float32),
            pltpu.SemaphoreType.DMA,
            pltpu.SemaphoreType.DMA,
        ],
    )
    return fn(src, dst, xl, xr, ea, att_flat, z128)


def _pass2_body(src_hbm, dst_hbm, ex_hbm, den_hbm, xl_hbm, z_hbm,
                out_hbm,
                src_v, dst_v, xl_v, ex_v, den_v, o_v, acc_sh,
                sem0, sem1):
    cid = lax.axis_index("c")
    sid = lax.axis_index("s")
    wid = cid * NS + sid

    pltpu.sync_copy(z_hbm.at[pl.ds(sid * ZR, ZR)], acc_sh.at[pl.ds(sid * ZR, ZR)])
    plsc.subcore_barrier()

    base0 = wid * EPT

    @pl.loop(0, NCHUNK)
    def _chunk(k):
        base = base0 + k * C
        pltpu.sync_copy(src_hbm.at[pl.ds(base, C)], src_v)
        pltpu.sync_copy(dst_hbm.at[pl.ds(base, C)], dst_v)
        cp1 = pltpu.async_copy(xl_hbm.at[src_v], xl_v, sem0)
        cp2 = pltpu.async_copy(den_hbm.at[dst_v], den_v, sem1)
        pltpu.sync_copy(ex_hbm.at[pl.ds(base, C)], ex_v)
        cp1.wait()
        cp2.wait()

        @pl.loop(0, C)
        def _edge(i):
            wv = ex_v[i, :] / (den_v[i, pl.ds(0, 16)] + 1e-16) * 0.25
            ws = [_bcast_lane(wv, h) for h in range(H)]
            for j in range(OUT // 16):
                o = ws[0] * xl_v[i, pl.ds(j * 16, 16)]
                for h in range(1, H):
                    o = o + ws[h] * xl_v[i, pl.ds(h * OUT + j * 16, 16)]
                o_v[i, pl.ds(j * 16, 16)] = o

        pltpu.sync_copy(o_v, acc_sh.at[dst_v], add=True)

    plsc.subcore_barrier()
    pltpu.sync_copy(acc_sh.at[pl.ds(sid * ZR, ZR)],
                    out_hbm.at[cid, pl.ds(sid * ZR, ZR)])


def _pass2(src, dst, ex, den, xl, z128):
    mesh = plsc.VectorSubcoreMesh(core_axis_name="c", subcore_axis_name="s")
    fn = pl.kernel(
        _pass2_body,
        out_type=jax.ShapeDtypeStruct((2, NP, OUT), jnp.float32),
        mesh=mesh,
        scratch_types=[
            pltpu.VMEM((C,), jnp.int32),
            pltpu.VMEM((C,), jnp.int32),
            pltpu.VMEM((C, HO), jnp.float32),
            pltpu.VMEM((C, 16), jnp.float32),
            pltpu.VMEM((C, OUT), jnp.float32),
            pltpu.VMEM((C, OUT), jnp.float32),
            pltpu.VMEM_SHARED((NP, OUT), jnp.float32),
            pltpu.SemaphoreType.DMA,
            pltpu.SemaphoreType.DMA,
        ],
    )
    return fn(src, dst, ex, den, xl, z128)


def kernel(x, edge_index, edge_attr, W_l, b_l, W_r, b_r, att, W_e, b_e, bias,
           gate_W, gate_b):
    del gate_W, gate_b

    src = jnp.concatenate([edge_index[0], jnp.zeros((EP - E,), jnp.int32)])
    dst = jnp.concatenate([edge_index[1], jnp.full((EP - E,), N, jnp.int32)])
    eap = jnp.concatenate([edge_attr, jnp.zeros((EPA - E, ED), jnp.float32)])
    xp = jnp.concatenate([x, jnp.zeros((NP - N, IN), jnp.float32)])
    att_flat = att.reshape(HO)
    z128 = jnp.zeros((NP, OUT), jnp.float32)

    xl, xr = _proj2(xp, W_l, b_l, W_r, b_r)
    ea = _proje(eap, W_e, b_e)
    ex, den_part = _pass1(src, dst, xl, xr, ea, att_flat, z128)
    den = _dencomb(den_part)
    out_part = _pass2(src, dst, ex, den, xl, z128)
    out = _final(out_part, bias)
    return out[:N]


# speedup vs baseline: 11.8308x; 1.3297x over previous
import jax
import jax.numpy as jnp
from jax import lax
from jax.experimental import pallas as pl
from jax.experimental.pallas import tpu as pltpu
from jax.experimental.pallas import tpu_sc as plsc

N = 10000
E = 320000
IN = 128
OUT = 128
H = 4
ED = 16
NEG = 0.2
HO = H * OUT

NP = 10240
NPD = NP // 8
EP = 321536
NT = 32
NS = 16
EPT = EP // NT
C = 32
NCHUNK = EPT // C
PAIRS = NCHUNK // 2
C2 = 16
NCHUNK2 = EPT // C2
PAIRS2 = NCHUNK2 // 2
ZR = NP // NS
ZRD = NPD // NS


def _proj2_body(x_ref, wl_ref, bl_ref, wr_ref, br_ref, xl_ref, xr_ref):
    xb = x_ref[...]
    xl_ref[...] = jnp.dot(xb, wl_ref[...], preferred_element_type=jnp.float32) + bl_ref[...]
    xr_ref[...] = jnp.dot(xb, wr_ref[...], preferred_element_type=jnp.float32) + br_ref[...]


def _proj2(xp, W_l, b_l, W_r, b_r):
    BM = 256
    grid = (NP // BM,)
    return pl.pallas_call(
        _proj2_body,
        grid=grid,
        in_specs=[
            pl.BlockSpec((BM, IN), lambda i: (i, 0)),
            pl.BlockSpec((IN, HO), lambda i: (0, 0)),
            pl.BlockSpec((1, HO), lambda i: (0, 0)),
            pl.BlockSpec((IN, HO), lambda i: (0, 0)),
            pl.BlockSpec((1, HO), lambda i: (0, 0)),
        ],
        out_specs=[
            pl.BlockSpec((BM, HO), lambda i: (i, 0)),
            pl.BlockSpec((BM, HO), lambda i: (i, 0)),
        ],
        out_shape=[
            jax.ShapeDtypeStruct((NP, HO), jnp.float32),
            jax.ShapeDtypeStruct((NP, HO), jnp.float32),
        ],
    )(xp, W_l, b_l.reshape(1, HO), W_r, b_r.reshape(1, HO))


def _proje_body(a_ref, w_ref, b_ref, o_ref):
    o_ref[...] = jnp.dot(a_ref[...], w_ref[...], preferred_element_type=jnp.float32) + b_ref[...]


def _proje(eap, W_e, b_e):
    BE = 1024
    grid = (EP // BE,)
    return pl.pallas_call(
        _proje_body,
        grid=grid,
        in_specs=[
            pl.BlockSpec((BE, ED), lambda i: (i, 0)),
            pl.BlockSpec((ED, HO), lambda i: (0, 0)),
            pl.BlockSpec((1, HO), lambda i: (0, 0)),
        ],
        out_specs=pl.BlockSpec((BE, HO), lambda i: (i, 0)),
        out_shape=jax.ShapeDtypeStruct((EP, HO), jnp.float32),
    )(eap, W_e, b_e.reshape(1, HO))


def _dencomb_body(dp_ref, o_ref):
    o_ref[...] = dp_ref[0] + dp_ref[1]


def _dencomb(den_part):
    return pl.pallas_call(
        _dencomb_body,
        out_shape=jax.ShapeDtypeStruct((NPD, OUT), jnp.float32),
    )(den_part)


def _final_body(p_ref, b_ref, o_ref):
    o_ref[...] = p_ref[0] + p_ref[1] + b_ref[...]


def _final(out_part, bias):
    return pl.pallas_call(
        _final_body,
        out_shape=jax.ShapeDtypeStruct((NP, OUT), jnp.float32),
    )(out_part, bias.reshape(1, OUT))


def _hsum(v):
    iot = lax.iota(jnp.int32, 16)
    for s in (8, 4, 2, 1):
        v = v + v.at[iot ^ s].get(mode="promise_in_bounds")
    return v


def _bcast_lane(v, lane):
    idx = jnp.full((16,), lane, jnp.int32)
    return v.at[idx].get(mode="promise_in_bounds")


def _elem(v0, v1, i):
    e0 = _bcast_lane(v0, jnp.minimum(i, 15))
    e1 = _bcast_lane(v1, jnp.maximum(i - 16, 0))
    return jnp.where(i < 16, e0, e1)


def _pass1_body(src_hbm, dst_hbm, xl_hbm, xr_hbm, ea_hbm, att_hbm, z_hbm,
                ex_hbm, den_hbm,
                src_v, dst_v, dstd_v, dstm_v, xl_v, xr_v, ea_v, ex_v, exd_v,
                att_v, den_sh, sems):
    cid = lax.axis_index("c")
    sid = lax.axis_index("s")
    wid = cid * NS + sid

    pltpu.sync_copy(z_hbm.at[pl.ds(sid * ZRD, ZRD)], den_sh.at[pl.ds(sid * ZRD, ZRD)])
    pltpu.sync_copy(att_hbm, att_v)
    plsc.subcore_barrier()

    iot = lax.iota(jnp.int32, 16)
    masks = [jnp.where(iot == h, 1.0, 0.0).astype(jnp.float32) for h in range(H)]
    base0 = wid * EPT

    def load_idx_and_start(b, k):
        base = base0 + k * C
        pltpu.sync_copy(src_hbm.at[pl.ds(base, C)], src_v.at[b])
        pltpu.sync_copy(dst_hbm.at[pl.ds(base, C)], dst_v.at[b])
        for j in range(C // 16):
            dw = dst_v[b, pl.ds(j * 16, 16)]
            dstd_v[b, pl.ds(j * 16, 16)] = lax.shift_right_logical(dw, 3)
            dstm_v[b, pl.ds(j * 16, 16)] = dw & 7
        pltpu.async_copy(xl_hbm.at[src_v.at[b]], xl_v.at[b], sems.at[b, 0])
        pltpu.async_copy(xr_hbm.at[dst_v.at[b]], xr_v.at[b], sems.at[b, 1])
        pltpu.async_copy(ea_hbm.at[pl.ds(base, C)], ea_v.at[b], sems.at[b, 2])

    def wait_bufs(b):
        pltpu.make_async_copy(xl_hbm.at[src_v.at[b]], xl_v.at[b], sems.at[b, 0]).wait()
        pltpu.make_async_copy(xr_hbm.at[dst_v.at[b]], xr_v.at[b], sems.at[b, 1]).wait()
        pltpu.make_async_copy(ea_hbm.at[pl.ds(0, C)], ea_v.at[b], sems.at[b, 2]).wait()

    load_idx_and_start(0, 0)
    load_idx_and_start(1, 1)

    @pl.loop(0, PAIRS)
    def _pair(k2):
        for b in (0, 1):
            k = k2 * 2 + b
            base = base0 + k * C
            wait_bufs(b)
            dm0 = dstm_v[b, pl.ds(0, 16)]
            dm1 = dstm_v[b, pl.ds(16, 16)]

            @pl.loop(0, C)
            def _edge(i):
                lvec = jnp.zeros((16,), jnp.float32)
                for h in range(H):
                    acc = jnp.zeros((16,), jnp.float32)
                    for j in range(OUT // 16):
                        off = h * OUT + j * 16
                        v = (xl_v[b, i, pl.ds(off, 16)] + xr_v[b, i, pl.ds(off, 16)]
                             + ea_v[b, i, pl.ds(off, 16)])
                        v = jnp.maximum(v, v * NEG)
                        acc = acc + v * att_v[pl.ds(off, 16)]
                    lvec = lvec + _hsum(acc) * masks[h]
                exv = jnp.exp(lvec)
                ex_v[b, i, :] = exv
                dstm = _elem(dm0, dm1, i)
                for j in range(8):
                    mj = jnp.where(dstm == j, 1.0, 0.0).astype(jnp.float32)
                    exd_v[i, pl.ds(j * 16, 16)] = exv * mj

            pltpu.sync_copy(ex_v.at[b], ex_hbm.at[pl.ds(base, C)])
            pltpu.sync_copy(exd_v, den_sh.at[dstd_v.at[b]], add=True)

            kp = jnp.minimum(k + 2, NCHUNK - 1)
            load_idx_and_start(b, kp)

    wait_bufs(0)
    wait_bufs(1)

    plsc.subcore_barrier()
    pltpu.sync_copy(den_sh.at[pl.ds(sid * ZRD, ZRD)],
                    den_hbm.at[cid, pl.ds(sid * ZRD, ZRD)])


def _pass1(src, dst, xl, xr, ea, att_flat, z128):
    mesh = plsc.VectorSubcoreMesh(core_axis_name="c", subcore_axis_name="s")
    fn = pl.kernel(
        _pass1_body,
        out_type=[
            jax.ShapeDtypeStruct((EP, 16), jnp.float32),
            jax.ShapeDtypeStruct((2, NPD, OUT), jnp.float32),
        ],
        mesh=mesh,
        scratch_types=[
            pltpu.VMEM((2, C), jnp.int32),
            pltpu.VMEM((2, C), jnp.int32),
            pltpu.VMEM((2, C), jnp.int32),
            pltpu.VMEM((2, C), jnp.int32),
            pltpu.VMEM((2, C, HO), jnp.float32),
            pltpu.VMEM((2, C, HO), jnp.float32),
            pltpu.VMEM((2, C, HO), jnp.float32),
            pltpu.VMEM((2, C, 16), jnp.float32),
            pltpu.VMEM((C, OUT), jnp.float32),
            pltpu.VMEM((HO,), jnp.float32),
            pltpu.VMEM_SHARED((NPD, OUT), jnp.float32),
            pltpu.SemaphoreType.DMA((2, 3)),
        ],
    )
    return fn(src, dst, xl, xr, ea, att_flat, z128)


def _pass2_body(src_hbm, dst_hbm, ex_hbm, den_hbm, xl_hbm, z_hbm,
                out_hbm,
                src_v, dst_v, dstd_v, dstm_v, xl_v, ex_v, den_v, o_v, acc_sh,
                sems):
    cid = lax.axis_index("c")
    sid = lax.axis_index("s")
    wid = cid * NS + sid

    pltpu.sync_copy(z_hbm.at[pl.ds(sid * ZR, ZR)], acc_sh.at[pl.ds(sid * ZR, ZR)])
    plsc.subcore_barrier()

    base0 = wid * EPT

    def load_idx_and_start(b, k):
        base = base0 + k * C2
        pltpu.sync_copy(src_hbm.at[pl.ds(base, C2)], src_v.at[b])
        pltpu.sync_copy(dst_hbm.at[pl.ds(base, C2)], dst_v.at[b])
        for j in range(C2 // 16):
            dw = dst_v[b, pl.ds(j * 16, 16)]
            dstd_v[b, pl.ds(j * 16, 16)] = lax.shift_right_logical(dw, 3)
            dstm_v[b, pl.ds(j * 16, 16)] = dw & 7
        pltpu.async_copy(xl_hbm.at[src_v.at[b]], xl_v.at[b], sems.at[b, 0])
        pltpu.async_copy(den_hbm.at[dstd_v.at[b]], den_v.at[b], sems.at[b, 1])
        pltpu.async_copy(ex_hbm.at[pl.ds(base, C2)], ex_v.at[b], sems.at[b, 2])

    def wait_bufs(b):
        pltpu.make_async_copy(xl_hbm.at[src_v.at[b]], xl_v.at[b], sems.at[b, 0]).wait()
        pltpu.make_async_copy(den_hbm.at[dstd_v.at[b]], den_v.at[b], sems.at[b, 1]).wait()
        pltpu.make_async_copy(ex_hbm.at[pl.ds(0, C2)], ex_v.at[b], sems.at[b, 2]).wait()

    load_idx_and_start(0, 0)
    load_idx_and_start(1, 1)

    @pl.loop(0, PAIRS2)
    def _pair(k2):
        for b in (0, 1):
            k = k2 * 2 + b
            wait_bufs(b)
            dm0 = dstm_v[b, pl.ds(0, 16)]

            @pl.loop(0, C2)
            def _edge(i):
                dstm = _bcast_lane(dm0, i)
                dv = jnp.zeros((16,), jnp.float32)
                for j in range(8):
                    mj = jnp.where(dstm == j, 1.0, 0.0).astype(jnp.float32)
                    dv = dv + den_v[b, i, pl.ds(j * 16, 16)] * mj
                wv = ex_v[b, i, :] / (dv + 1e-16) * 0.25
                ws = [_bcast_lane(wv, h) for h in range(H)]
                for j in range(OUT // 16):
                    o = ws[0] * xl_v[b, i, pl.ds(j * 16, 16)]
                    for h in range(1, H):
                        o = o + ws[h] * xl_v[b, i, pl.ds(h * OUT + j * 16, 16)]
                    o_v[i, pl.ds(j * 16, 16)] = o

            pltpu.sync_copy(o_v, acc_sh.at[dst_v.at[b]], add=True)

            kp = jnp.minimum(k + 2, NCHUNK2 - 1)
            load_idx_and_start(b, kp)

    wait_bufs(0)
    wait_bufs(1)

    plsc.subcore_barrier()
    pltpu.sync_copy(acc_sh.at[pl.ds(sid * ZR, ZR)],
                    out_hbm.at[cid, pl.ds(sid * ZR, ZR)])


def _pass2(src, dst, ex, den, xl, z128):
    mesh = plsc.VectorSubcoreMesh(core_axis_name="c", subcore_axis_name="s")
    fn = pl.kernel(
        _pass2_body,
        out_type=jax.ShapeDtypeStruct((2, NP, OUT), jnp.float32),
        mesh=mesh,
        scratch_types=[
            pltpu.VMEM((2, C2), jnp.int32),
            pltpu.VMEM((2, C2), jnp.int32),
            pltpu.VMEM((2, C2), jnp.int32),
            pltpu.VMEM((2, C2), jnp.int32),
            pltpu.VMEM((2, C2, HO), jnp.float32),
            pltpu.VMEM((2, C2, 16), jnp.float32),
            pltpu.VMEM((2, C2, OUT), jnp.float32),
            pltpu.VMEM((C2, OUT), jnp.float32),
            pltpu.VMEM_SHARED((NP, OUT), jnp.float32),
            pltpu.SemaphoreType.DMA((2, 3)),
        ],
    )
    return fn(src, dst, ex, den, xl, z128)


def kernel(x, edge_index, edge_attr, W_l, b_l, W_r, b_r, att, W_e, b_e, bias,
           gate_W, gate_b):
    del gate_W, gate_b

    src = jnp.concatenate([edge_index[0], jnp.zeros((EP - E,), jnp.int32)])
    dst = jnp.concatenate([edge_index[1], jnp.full((EP - E,), N, jnp.int32)])
    eap = jnp.concatenate([edge_attr, jnp.zeros((EP - E, ED), jnp.float32)])
    xp = jnp.concatenate([x, jnp.zeros((NP - N, IN), jnp.float32)])
    att_flat = att.reshape(HO)
    z128 = jnp.zeros((NP, OUT), jnp.float32)

    xl, xr = _proj2(xp, W_l, b_l, W_r, b_r)
    ea = _proje(eap, W_e, b_e)
    ex, den_part = _pass1(src, dst, xl, xr, ea, att_flat, z128)
    den = _dencomb(den_part)
    out_part = _pass2(src, dst, ex, den, xl, z128)
    out = _final(out_part, bias)
    return out[:N]
